# baseline (speedup 1.0000x reference)
import functools
import math

import jax
import jax.numpy as jnp
from jax import lax
from jax.experimental import pallas as pl
from jax.experimental.pallas import tpu as pltpu
from jax.experimental.pallas import tpu_sc as plsc

EMBED_DIM = 64
HIDDEN_DIM = 128
U_GROUP = 101
I_GROUP = 101
RECENT = 50
TAU = 0.8
BATCH_BLK = 256

_NC = 2
_NS = 16
_NW = _NC * _NS
_GW = 128
_PAD = 64
_SPS = 8

_HIGH = jax.lax.Precision.HIGHEST


def _dot(a, b):
    return jnp.dot(a, b, preferred_element_type=jnp.float32)


@jax.jit
def _sc_gather(user_embed, item_embed, uid, iid, ibu, ibi):
    B = uid.shape[0]
    NR = ibu.shape[0]
    mesh = plsc.VectorSubcoreMesh(core_axis_name="c", subcore_axis_name="s")
    out_type = (
        jax.ShapeDtypeStruct((B, EMBED_DIM), jnp.float32),
        jax.ShapeDtypeStruct((B, EMBED_DIM), jnp.float32),
        jax.ShapeDtypeStruct((NR, EMBED_DIM), jnp.float32),
        jax.ShapeDtypeStruct((NR, EMBED_DIM), jnp.float32),
    )

    @functools.partial(
        pl.kernel,
        mesh=mesh,
        out_type=out_type,
        compiler_params=pltpu.CompilerParams(use_tc_tiling_on_sc=False),
        scratch_types=[
            pltpu.VMEM((_GW,), jnp.int32),
            pltpu.VMEM((_GW,), jnp.int32),
            pltpu.VMEM((_GW, EMBED_DIM), jnp.float32),
            pltpu.VMEM((_GW, EMBED_DIM), jnp.float32),
            pltpu.SemaphoreType.DMA,
            pltpu.SemaphoreType.DMA,
            pltpu.SemaphoreType.DMA,
            pltpu.SemaphoreType.DMA,
        ],
    )
    def k(ue_hbm, ie_hbm, uid_hbm, iid_hbm, ibu_hbm, ibi_hbm,
          o_ue, o_ie, o_ur, o_ir,
          idx0, idx1, rows0, rows1, g0, g1, w0, w1):
        wid = lax.axis_index("s") * _NC + lax.axis_index("c")

        def job(idx_hbm, table_hbm, out_hbm, n):
            per_w = n // _NW
            chunks = per_w // _GW
            base0 = wid * per_w

            @pl.loop(0, chunks)
            def _(c):
                base = base0 + c * _GW
                pltpu.sync_copy(idx_hbm.at[pl.ds(base, _GW)], idx0)
                pltpu.async_copy(table_hbm.at[idx0], rows0, g0).wait()
                pltpu.sync_copy(rows0, out_hbm.at[pl.ds(base, _GW)])

        def id_job(idx_hbm, table_hbm, out_hbm):
            base = wid * (B // _NW)
            pltpu.sync_copy(idx_hbm.at[pl.ds(base, _GW)], idx0)
            pltpu.async_copy(table_hbm.at[idx0], rows0, g0).wait()
            pltpu.sync_copy(rows0, out_hbm.at[pl.ds(base, _GW)])

        id_job(uid_hbm, ue_hbm, o_ue)
        id_job(iid_hbm, ie_hbm, o_ie)
        job(ibu_hbm, ie_hbm, o_ur, NR)
        job(ibi_hbm, ue_hbm, o_ir, NR)

    return k(user_embed, item_embed, uid, iid, ibu, ibi)


def _softmax_rows(counts):
    e = jnp.exp(jnp.log(counts + 1.0) / TAU)
    return e / jnp.sum(e, axis=1, keepdims=True)


def _stage1_body(cu1, cu2, ci1, ci2, uemb, iemb, gur, gir, ibu, ibi,
                 ie0, ue0, um, im, uV, iV, W1, b1, o_ref):
    mVu = _dot(um[...], uV[...])
    mVi = _dot(im[...], iV[...])

    u1 = _dot(_softmax_rows(cu1[...]), mVi)
    u2 = _dot(_softmax_rows(cu2[...]), mVu)
    i1 = _dot(_softmax_rows(ci1[...]), mVu)
    i2 = _dot(_softmax_rows(ci2[...]), mVi)

    n0u = jnp.sum((ibu[...] == 0).astype(jnp.float32), axis=1)[:, None]
    n0i = jnp.sum((ibi[...] == 0).astype(jnp.float32), axis=1)[:, None]
    gu = gur[...].reshape(BATCH_BLK, _PAD // 2, 2 * EMBED_DIM)
    gi = gir[...].reshape(BATCH_BLK, _PAD // 2, 2 * EMBED_DIM)
    su2 = jnp.sum(gu, axis=1)
    si2 = jnp.sum(gi, axis=1)
    s_ur = su2[:, :EMBED_DIM] + su2[:, EMBED_DIM:] - n0u * ie0[...]
    s_ir = si2[:, :EMBED_DIM] + si2[:, EMBED_DIM:] - n0i * ue0[...]
    urw = _dot(s_ur, iV[...])
    irw = _dot(s_ir, uV[...])

    h1 = (_dot(uemb[...], W1[0:64, :])
          + _dot(u1, W1[64:192, :])
          + _dot(u2, W1[192:320, :])
          + _dot(urw, W1[320:448, :])
          + _dot(iemb[...], W1[448:512, :])
          + _dot(i1, W1[512:640, :])
          + _dot(i2, W1[640:768, :])
          + _dot(irw, W1[768:896, :])
          + b1[...])
    o_ref[...] = h1


def _stage1(cu1, cu2, ci1, ci2, uemb, iemb, gur, gir, ibu, ibi,
            ie0, ue0, um, im, uV, iV, W1, b1):
    B = cu1.shape[0]
    nblk = B // BATCH_BLK
    D1 = W1.shape[1]
    blk = lambda s: pl.BlockSpec(s, lambda i: (i,) + (0,) * (len(s) - 1))
    full2 = lambda a: pl.BlockSpec(a.shape, lambda i: (0,) * a.ndim)
    g_spec = pl.BlockSpec((BATCH_BLK * _PAD // 2, 2 * EMBED_DIM),
                          lambda i: (i, 0))
    return pl.pallas_call(
        _stage1_body,
        grid=(nblk,),
        in_specs=[
            blk((BATCH_BLK, I_GROUP)), blk((BATCH_BLK, U_GROUP)),
            blk((BATCH_BLK, U_GROUP)), blk((BATCH_BLK, I_GROUP)),
            blk((BATCH_BLK, EMBED_DIM)), blk((BATCH_BLK, EMBED_DIM)),
            g_spec, g_spec,
            blk((BATCH_BLK, _PAD)), blk((BATCH_BLK, _PAD)),
            full2(ie0), full2(ue0), full2(um), full2(im),
            full2(uV), full2(iV), full2(W1), full2(b1),
        ],
        out_specs=blk((BATCH_BLK, D1)),
        out_shape=jax.ShapeDtypeStruct((B, D1), jnp.float32),
    )(cu1, cu2, ci1, ci2, uemb, iemb, gur, gir, ibu, ibi,
      ie0, ue0, um, im, uV, iV, W1, b1)


def _dice(h, alpha):
    n = h.shape[0]
    avg = jnp.mean(h, axis=0, keepdims=True)
    d = h - avg
    std = jnp.sqrt(jnp.sum(d * d, axis=0, keepdims=True) / (n - 1))
    p = jax.nn.sigmoid(d / std + 1e-08)
    return h * p + alpha * h * (1 - p)


def _stage2_body(h1, a1, W2, b2, a2, W3, b3, o_ref):
    h = _dice(h1[...], a1[0, 0])
    h = _dice(_dot(h, W2[...]) + b2[...], a2[0, 0])
    o_ref[...] = jax.nn.sigmoid(_dot(h, W3[...]) + b3[...])


def _stage2(h1, a1, W2, b2, a2, W3, b3):
    B = h1.shape[0]
    return pl.pallas_call(
        _stage2_body,
        out_shape=jax.ShapeDtypeStruct((B, 1), jnp.float32),
    )(h1, a1, W2, b2, a2, W3, b3)


def kernel(x, user_embed, item_embed, u_macro_embed, i_macro_embed,
           uQ, uK, uV, iQ, iK, iV, W1, b1, alpha1, W2, b2, alpha2, W3, b3):
    f32 = jnp.float32
    x = x.astype(jnp.int32)
    uid = x[:, 0]
    cu1 = x[:, 1:1 + I_GROUP].astype(f32)
    cu2 = x[:, 1 + I_GROUP:1 + I_GROUP + U_GROUP].astype(f32)
    ur = x[:, 1 + I_GROUP + U_GROUP:1 + I_GROUP + U_GROUP + RECENT]
    ic = 1 + I_GROUP + U_GROUP + RECENT
    iid = x[:, ic]
    ci1 = x[:, ic + 1:ic + 1 + U_GROUP].astype(f32)
    ci2 = x[:, ic + 1 + U_GROUP:ic + 1 + U_GROUP + I_GROUP].astype(f32)
    ir = x[:, ic + 1 + U_GROUP + I_GROUP:]

    B = x.shape[0]
    z14 = jnp.zeros((B, _PAD - RECENT), jnp.int32)
    ibu = jnp.concatenate([ur, z14], axis=1)
    ibi = jnp.concatenate([ir, z14], axis=1)

    uemb, iemb, g_ur, g_ir = _sc_gather(user_embed, item_embed, uid, iid,
                                        ibu.reshape(-1), ibi.reshape(-1))
    g_ur = g_ur.reshape(-1, 2 * EMBED_DIM)
    g_ir = g_ir.reshape(-1, 2 * EMBED_DIM)

    h1 = _stage1(cu1, cu2, ci1, ci2, uemb, iemb, g_ur, g_ir, ibu, ibi,
                 item_embed[0:1, :], user_embed[0:1, :],
                 u_macro_embed, i_macro_embed, uV, iV,
                 W1, b1.reshape(1, -1))
    return _stage2(h1, alpha1.reshape(1, 1), W2, b2.reshape(1, -1),
                   alpha2.reshape(1, 1), W3, b3.reshape(1, 1))


# speedup vs baseline: 5.7723x; 5.7723x over previous
import functools
import math

import jax
import jax.numpy as jnp
from jax import lax
from jax.experimental import pallas as pl
from jax.experimental.pallas import tpu as pltpu
from jax.experimental.pallas import tpu_sc as plsc

EMBED_DIM = 64
HIDDEN_DIM = 128
U_GROUP = 101
I_GROUP = 101
RECENT = 50
TAU = 0.8
BATCH_BLK = 256

_NC = 2
_NS = 16
_NW = _NC * _NS
_GW = 128
_PAD = 64
_SPS = 8

_HIGH = jax.lax.Precision.HIGHEST


def _dot(a, b):
    return jnp.dot(a, b, preferred_element_type=jnp.float32)


@jax.jit
def _sc_gather(user_embed, item_embed, uid, iid, ibu, ibi):
    B = uid.shape[0]
    NR = ibu.shape[0]
    mesh = plsc.VectorSubcoreMesh(core_axis_name="c", subcore_axis_name="s")
    out_type = (
        jax.ShapeDtypeStruct((B, EMBED_DIM), jnp.float32),
        jax.ShapeDtypeStruct((B, EMBED_DIM), jnp.float32),
        jax.ShapeDtypeStruct((NR, EMBED_DIM), jnp.float32),
        jax.ShapeDtypeStruct((NR, EMBED_DIM), jnp.float32),
    )

    @functools.partial(
        pl.kernel,
        mesh=mesh,
        out_type=out_type,
        compiler_params=pltpu.CompilerParams(use_tc_tiling_on_sc=False),
        scratch_types=[
            pltpu.VMEM((_GW,), jnp.int32),
            pltpu.VMEM((_GW,), jnp.int32),
            pltpu.VMEM((_GW, EMBED_DIM), jnp.float32),
            pltpu.VMEM((_GW, EMBED_DIM), jnp.float32),
            pltpu.SemaphoreType.DMA,
            pltpu.SemaphoreType.DMA,
            pltpu.SemaphoreType.DMA,
            pltpu.SemaphoreType.DMA,
        ],
    )
    def k(ue_hbm, ie_hbm, uid_hbm, iid_hbm, ibu_hbm, ibi_hbm,
          o_ue, o_ie, o_ur, o_ir,
          idx0, idx1, rows0, rows1, g0, g1, w0, w1):
        wid = lax.axis_index("s") * _NC + lax.axis_index("c")

        def job(idx_hbm, table_hbm, out_hbm, n):
            per_w = n // _NW
            chunks = per_w // _GW
            base0 = wid * per_w

            @pl.loop(0, chunks)
            def _(c):
                base = base0 + c * _GW
                pltpu.sync_copy(idx_hbm.at[pl.ds(base, _GW)], idx0)
                pltpu.async_copy(table_hbm.at[idx0], rows0, g0).wait()
                pltpu.sync_copy(rows0, out_hbm.at[pl.ds(base, _GW)])

        def id_job(idx_hbm, table_hbm, out_hbm):
            base = wid * (B // _NW)
            pltpu.sync_copy(idx_hbm.at[pl.ds(base, _GW)], idx0)
            pltpu.async_copy(table_hbm.at[idx0], rows0, g0).wait()
            pltpu.sync_copy(rows0, out_hbm.at[pl.ds(base, _GW)])

        id_job(uid_hbm, ue_hbm, o_ue)
        id_job(iid_hbm, ie_hbm, o_ie)
        job(ibu_hbm, ie_hbm, o_ur, NR)
        job(ibi_hbm, ue_hbm, o_ir, NR)

    return k(user_embed, item_embed, uid, iid, ibu, ibi)


def _softmax_rows(counts):
    e = jnp.exp(jnp.log(counts + 1.0) / TAU)
    return e / jnp.sum(e, axis=1, keepdims=True)


def _stage1_body(cu1, cu2, ci1, ci2, uemb, iemb, gur, gir, ibu, ibi,
                 ie0, ue0, um, im, uV, iV, W1, b1, o_ref):
    mVu = _dot(um[...], uV[...])
    mVi = _dot(im[...], iV[...])

    u1 = _dot(_softmax_rows(cu1[...]), mVi)
    u2 = _dot(_softmax_rows(cu2[...]), mVu)
    i1 = _dot(_softmax_rows(ci1[...]), mVu)
    i2 = _dot(_softmax_rows(ci2[...]), mVi)

    n0u = jnp.sum((ibu[...][:, :RECENT] == 0).astype(jnp.float32),
                  axis=1)[:, None]
    n0i = jnp.sum((ibi[...][:, :RECENT] == 0).astype(jnp.float32),
                  axis=1)[:, None]
    gu = gur[...].reshape(BATCH_BLK, _PAD // 2, 2 * EMBED_DIM)
    gi = gir[...].reshape(BATCH_BLK, _PAD // 2, 2 * EMBED_DIM)
    su2 = jnp.sum(gu[:, :RECENT // 2, :], axis=1)
    si2 = jnp.sum(gi[:, :RECENT // 2, :], axis=1)
    s_ur = su2[:, :EMBED_DIM] + su2[:, EMBED_DIM:] - n0u * ie0[...]
    s_ir = si2[:, :EMBED_DIM] + si2[:, EMBED_DIM:] - n0i * ue0[...]
    urw = _dot(s_ur, iV[...])
    irw = _dot(s_ir, uV[...])

    h1 = (_dot(uemb[...], W1[0:64, :])
          + _dot(u1, W1[64:192, :])
          + _dot(u2, W1[192:320, :])
          + _dot(urw, W1[320:448, :])
          + _dot(iemb[...], W1[448:512, :])
          + _dot(i1, W1[512:640, :])
          + _dot(i2, W1[640:768, :])
          + _dot(irw, W1[768:896, :])
          + b1[...])
    o_ref[...] = h1


def _stage1(cu1, cu2, ci1, ci2, uemb, iemb, gur, gir, ibu, ibi,
            ie0, ue0, um, im, uV, iV, W1, b1):
    B = cu1.shape[0]
    nblk = B // BATCH_BLK
    D1 = W1.shape[1]
    blk = lambda s: pl.BlockSpec(s, lambda i: (i,) + (0,) * (len(s) - 1))
    full2 = lambda a: pl.BlockSpec(a.shape, lambda i: (0,) * a.ndim)
    g_spec = pl.BlockSpec((BATCH_BLK * _PAD // 2, 2 * EMBED_DIM),
                          lambda i: (i, 0))
    return pl.pallas_call(
        _stage1_body,
        grid=(nblk,),
        in_specs=[
            blk((BATCH_BLK, I_GROUP)), blk((BATCH_BLK, U_GROUP)),
            blk((BATCH_BLK, U_GROUP)), blk((BATCH_BLK, I_GROUP)),
            blk((BATCH_BLK, EMBED_DIM)), blk((BATCH_BLK, EMBED_DIM)),
            g_spec, g_spec,
            blk((BATCH_BLK, _PAD)), blk((BATCH_BLK, _PAD)),
            full2(ie0), full2(ue0), full2(um), full2(im),
            full2(uV), full2(iV), full2(W1), full2(b1),
        ],
        out_specs=blk((BATCH_BLK, D1)),
        out_shape=jax.ShapeDtypeStruct((B, D1), jnp.float32),
    )(cu1, cu2, ci1, ci2, uemb, iemb, gur, gir, ibu, ibi,
      ie0, ue0, um, im, uV, iV, W1, b1)


def _dice(h, alpha):
    n = h.shape[0]
    avg = jnp.mean(h, axis=0, keepdims=True)
    d = h - avg
    std = jnp.sqrt(jnp.sum(d * d, axis=0, keepdims=True) / (n - 1))
    p = jax.nn.sigmoid(d / std + 1e-08)
    return h * p + alpha * h * (1 - p)


def _stage2_body(h1, a1, W2, b2, a2, W3, b3, o_ref):
    h = _dice(h1[...], a1[0, 0])
    h = _dice(_dot(h, W2[...]) + b2[...], a2[0, 0])
    o_ref[...] = jax.nn.sigmoid(_dot(h, W3[...]) + b3[...])


def _stage2(h1, a1, W2, b2, a2, W3, b3):
    B = h1.shape[0]
    return pl.pallas_call(
        _stage2_body,
        out_shape=jax.ShapeDtypeStruct((B, 1), jnp.float32),
    )(h1, a1, W2, b2, a2, W3, b3)


def kernel(x, user_embed, item_embed, u_macro_embed, i_macro_embed,
           uQ, uK, uV, iQ, iK, iV, W1, b1, alpha1, W2, b2, alpha2, W3, b3):
    f32 = jnp.float32
    x = x.astype(jnp.int32)
    uid = x[:, 0]
    cu1 = x[:, 1:1 + I_GROUP].astype(f32)
    cu2 = x[:, 1 + I_GROUP:1 + I_GROUP + U_GROUP].astype(f32)
    ur = x[:, 1 + I_GROUP + U_GROUP:1 + I_GROUP + U_GROUP + RECENT]
    ic = 1 + I_GROUP + U_GROUP + RECENT
    iid = x[:, ic]
    ci1 = x[:, ic + 1:ic + 1 + U_GROUP].astype(f32)
    ci2 = x[:, ic + 1 + U_GROUP:ic + 1 + U_GROUP + I_GROUP].astype(f32)
    ir = x[:, ic + 1 + U_GROUP + I_GROUP:]

    B = x.shape[0]
    npad = _PAD - RECENT
    ibu = jnp.concatenate([ur, ur[:, :npad]], axis=1)
    ibi = jnp.concatenate([ir, ir[:, :npad]], axis=1)

    uemb, iemb, g_ur, g_ir = _sc_gather(user_embed, item_embed, uid, iid,
                                        ibu.reshape(-1), ibi.reshape(-1))
    g_ur = g_ur.reshape(-1, 2 * EMBED_DIM)
    g_ir = g_ir.reshape(-1, 2 * EMBED_DIM)

    h1 = _stage1(cu1, cu2, ci1, ci2, uemb, iemb, g_ur, g_ir, ibu, ibi,
                 item_embed[0:1, :], user_embed[0:1, :],
                 u_macro_embed, i_macro_embed, uV, iV,
                 W1, b1.reshape(1, -1))
    return _stage2(h1, alpha1.reshape(1, 1), W2, b2.reshape(1, -1),
                   alpha2.reshape(1, 1), W3, b3.reshape(1, 1))


# speedup vs baseline: 6.9466x; 1.2035x over previous
import functools
import math

import jax
import jax.numpy as jnp
from jax import lax
from jax.experimental import pallas as pl
from jax.experimental.pallas import tpu as pltpu
from jax.experimental.pallas import tpu_sc as plsc

EMBED_DIM = 64
HIDDEN_DIM = 128
U_GROUP = 101
I_GROUP = 101
RECENT = 50
TAU = 0.8
BATCH_BLK = 256

_NC = 2
_NS = 16
_NW = _NC * _NS
_GW = 128
_PAD = 64
_SPS = 8

_HIGH = jax.lax.Precision.HIGHEST


def _dot(a, b):
    return jnp.dot(a, b, preferred_element_type=jnp.float32)


@jax.jit
def _sc_gather(user_embed, item_embed, uid, iid, ibu, ibi):
    B = uid.shape[0]
    NR = ibu.shape[0]
    mesh = plsc.VectorSubcoreMesh(core_axis_name="c", subcore_axis_name="s")
    out_type = (
        jax.ShapeDtypeStruct((B, EMBED_DIM), jnp.float32),
        jax.ShapeDtypeStruct((B, EMBED_DIM), jnp.float32),
        jax.ShapeDtypeStruct((NR, EMBED_DIM), jnp.float32),
        jax.ShapeDtypeStruct((NR, EMBED_DIM), jnp.float32),
    )

    @functools.partial(
        pl.kernel,
        mesh=mesh,
        out_type=out_type,
        compiler_params=pltpu.CompilerParams(use_tc_tiling_on_sc=False),
        scratch_types=[
            pltpu.VMEM((_GW,), jnp.int32),
            pltpu.VMEM((_GW,), jnp.int32),
            pltpu.VMEM((_GW, EMBED_DIM), jnp.float32),
            pltpu.VMEM((_GW, EMBED_DIM), jnp.float32),
            pltpu.SemaphoreType.DMA,
            pltpu.SemaphoreType.DMA,
            pltpu.SemaphoreType.DMA,
            pltpu.SemaphoreType.DMA,
        ],
    )
    def k(ue_hbm, ie_hbm, uid_hbm, iid_hbm, ibu_hbm, ibi_hbm,
          o_ue, o_ie, o_ur, o_ir,
          idx0, idx1, rows0, rows1, g0, g1, w0, w1):
        wid = lax.axis_index("s") * _NC + lax.axis_index("c")

        def job(idx_hbm, table_hbm, out_hbm, n):
            per_w = n // _NW
            chunks = per_w // _GW
            base0 = wid * per_w

            @pl.loop(0, chunks // 2)
            def _(t):
                ba = base0 + (2 * t) * _GW
                bb = ba + _GW
                pltpu.sync_copy(idx_hbm.at[pl.ds(ba, _GW)], idx0)
                ga = pltpu.async_copy(table_hbm.at[idx0], rows0, g0)
                pltpu.sync_copy(idx_hbm.at[pl.ds(bb, _GW)], idx1)
                gb = pltpu.async_copy(table_hbm.at[idx1], rows1, g1)
                ga.wait()
                wa = pltpu.async_copy(rows0, out_hbm.at[pl.ds(ba, _GW)], w0)
                gb.wait()
                wb = pltpu.async_copy(rows1, out_hbm.at[pl.ds(bb, _GW)], w1)
                wa.wait()
                wb.wait()

        def id_job(idx_hbm, table_hbm, out_hbm):
            base = wid * (B // _NW)
            pltpu.sync_copy(idx_hbm.at[pl.ds(base, _GW)], idx0)
            pltpu.async_copy(table_hbm.at[idx0], rows0, g0).wait()
            pltpu.sync_copy(rows0, out_hbm.at[pl.ds(base, _GW)])

        id_job(uid_hbm, ue_hbm, o_ue)
        id_job(iid_hbm, ie_hbm, o_ie)
        job(ibu_hbm, ie_hbm, o_ur, NR)
        job(ibi_hbm, ue_hbm, o_ir, NR)

    return k(user_embed, item_embed, uid, iid, ibu, ibi)


def _softmax_rows(counts):
    e = jnp.exp(jnp.log(counts + 1.0) / TAU)
    return e / jnp.sum(e, axis=1, keepdims=True)


def _stage1_body(cu1, cu2, ci1, ci2, uemb, iemb, gur, gir, ibu, ibi,
                 ie0, ue0, um, im, uV, iV, W1, b1, o_ref):
    mVu = _dot(um[...], uV[...])
    mVi = _dot(im[...], iV[...])

    u1 = _dot(_softmax_rows(cu1[...]), mVi)
    u2 = _dot(_softmax_rows(cu2[...]), mVu)
    i1 = _dot(_softmax_rows(ci1[...]), mVu)
    i2 = _dot(_softmax_rows(ci2[...]), mVi)

    n0u = jnp.sum((ibu[...][:, :RECENT] == 0).astype(jnp.float32),
                  axis=1)[:, None]
    n0i = jnp.sum((ibi[...][:, :RECENT] == 0).astype(jnp.float32),
                  axis=1)[:, None]
    gu = gur[...].reshape(BATCH_BLK, _PAD // 2, 2 * EMBED_DIM)
    gi = gir[...].reshape(BATCH_BLK, _PAD // 2, 2 * EMBED_DIM)
    su2 = jnp.sum(gu[:, :RECENT // 2, :], axis=1)
    si2 = jnp.sum(gi[:, :RECENT // 2, :], axis=1)
    s_ur = su2[:, :EMBED_DIM] + su2[:, EMBED_DIM:] - n0u * ie0[...]
    s_ir = si2[:, :EMBED_DIM] + si2[:, EMBED_DIM:] - n0i * ue0[...]
    urw = _dot(s_ur, iV[...])
    irw = _dot(s_ir, uV[...])

    h1 = (_dot(uemb[...], W1[0:64, :])
          + _dot(u1, W1[64:192, :])
          + _dot(u2, W1[192:320, :])
          + _dot(urw, W1[320:448, :])
          + _dot(iemb[...], W1[448:512, :])
          + _dot(i1, W1[512:640, :])
          + _dot(i2, W1[640:768, :])
          + _dot(irw, W1[768:896, :])
          + b1[...])
    o_ref[...] = h1


def _stage1(cu1, cu2, ci1, ci2, uemb, iemb, gur, gir, ibu, ibi,
            ie0, ue0, um, im, uV, iV, W1, b1):
    B = cu1.shape[0]
    nblk = B // BATCH_BLK
    D1 = W1.shape[1]
    blk = lambda s: pl.BlockSpec(s, lambda i: (i,) + (0,) * (len(s) - 1))
    full2 = lambda a: pl.BlockSpec(a.shape, lambda i: (0,) * a.ndim)
    g_spec = pl.BlockSpec((BATCH_BLK * _PAD // 2, 2 * EMBED_DIM),
                          lambda i: (i, 0))
    return pl.pallas_call(
        _stage1_body,
        grid=(nblk,),
        in_specs=[
            blk((BATCH_BLK, I_GROUP)), blk((BATCH_BLK, U_GROUP)),
            blk((BATCH_BLK, U_GROUP)), blk((BATCH_BLK, I_GROUP)),
            blk((BATCH_BLK, EMBED_DIM)), blk((BATCH_BLK, EMBED_DIM)),
            g_spec, g_spec,
            blk((BATCH_BLK, _PAD)), blk((BATCH_BLK, _PAD)),
            full2(ie0), full2(ue0), full2(um), full2(im),
            full2(uV), full2(iV), full2(W1), full2(b1),
        ],
        out_specs=blk((BATCH_BLK, D1)),
        out_shape=jax.ShapeDtypeStruct((B, D1), jnp.float32),
    )(cu1, cu2, ci1, ci2, uemb, iemb, gur, gir, ibu, ibi,
      ie0, ue0, um, im, uV, iV, W1, b1)


def _dice(h, alpha):
    n = h.shape[0]
    avg = jnp.mean(h, axis=0, keepdims=True)
    d = h - avg
    std = jnp.sqrt(jnp.sum(d * d, axis=0, keepdims=True) / (n - 1))
    p = jax.nn.sigmoid(d / std + 1e-08)
    return h * p + alpha * h * (1 - p)


def _stage2_body(h1, a1, W2, b2, a2, W3, b3, o_ref):
    h = _dice(h1[...], a1[0, 0])
    h = _dice(_dot(h, W2[...]) + b2[...], a2[0, 0])
    o_ref[...] = jax.nn.sigmoid(_dot(h, W3[...]) + b3[...])


def _stage2(h1, a1, W2, b2, a2, W3, b3):
    B = h1.shape[0]
    return pl.pallas_call(
        _stage2_body,
        out_shape=jax.ShapeDtypeStruct((B, 1), jnp.float32),
    )(h1, a1, W2, b2, a2, W3, b3)


def kernel(x, user_embed, item_embed, u_macro_embed, i_macro_embed,
           uQ, uK, uV, iQ, iK, iV, W1, b1, alpha1, W2, b2, alpha2, W3, b3):
    f32 = jnp.float32
    x = x.astype(jnp.int32)
    uid = x[:, 0]
    cu1 = x[:, 1:1 + I_GROUP].astype(f32)
    cu2 = x[:, 1 + I_GROUP:1 + I_GROUP + U_GROUP].astype(f32)
    ur = x[:, 1 + I_GROUP + U_GROUP:1 + I_GROUP + U_GROUP + RECENT]
    ic = 1 + I_GROUP + U_GROUP + RECENT
    iid = x[:, ic]
    ci1 = x[:, ic + 1:ic + 1 + U_GROUP].astype(f32)
    ci2 = x[:, ic + 1 + U_GROUP:ic + 1 + U_GROUP + I_GROUP].astype(f32)
    ir = x[:, ic + 1 + U_GROUP + I_GROUP:]

    B = x.shape[0]
    npad = _PAD - RECENT
    ibu = jnp.concatenate([ur, ur[:, :npad]], axis=1)
    ibi = jnp.concatenate([ir, ir[:, :npad]], axis=1)

    uemb, iemb, g_ur, g_ir = _sc_gather(user_embed, item_embed, uid, iid,
                                        ibu.reshape(-1), ibi.reshape(-1))
    g_ur = g_ur.reshape(-1, 2 * EMBED_DIM)
    g_ir = g_ir.reshape(-1, 2 * EMBED_DIM)

    h1 = _stage1(cu1, cu2, ci1, ci2, uemb, iemb, g_ur, g_ir, ibu, ibi,
                 item_embed[0:1, :], user_embed[0:1, :],
                 u_macro_embed, i_macro_embed, uV, iV,
                 W1, b1.reshape(1, -1))
    return _stage2(h1, alpha1.reshape(1, 1), W2, b2.reshape(1, -1),
                   alpha2.reshape(1, 1), W3, b3.reshape(1, 1))


# speedup vs baseline: 8.3454x; 1.2014x over previous
import functools
import math

import jax
import jax.numpy as jnp
from jax import lax
from jax.experimental import pallas as pl
from jax.experimental.pallas import tpu as pltpu
from jax.experimental.pallas import tpu_sc as plsc

EMBED_DIM = 64
HIDDEN_DIM = 128
U_GROUP = 101
I_GROUP = 101
RECENT = 50
TAU = 0.8
BATCH_BLK = 256

_NC = 2
_NS = 16
_NW = _NC * _NS
_GW = 128
_PAD = 64
_SPS = 8

_HIGH = jax.lax.Precision.HIGHEST


def _dot(a, b):
    return jnp.dot(a, b, preferred_element_type=jnp.float32)


@jax.jit
def _sc_gather(user_embed, item_embed, uid, iid, ibu, ibi):
    B = uid.shape[0]
    NR = ibu.shape[0]
    mesh = plsc.VectorSubcoreMesh(core_axis_name="c", subcore_axis_name="s")
    out_type = (
        jax.ShapeDtypeStruct((B, EMBED_DIM), jnp.float32),
        jax.ShapeDtypeStruct((B, EMBED_DIM), jnp.float32),
        jax.ShapeDtypeStruct((NR, EMBED_DIM), jnp.float32),
        jax.ShapeDtypeStruct((NR, EMBED_DIM), jnp.float32),
    )

    @functools.partial(
        pl.kernel,
        mesh=mesh,
        out_type=out_type,
        compiler_params=pltpu.CompilerParams(use_tc_tiling_on_sc=False),
        scratch_types=[
            pltpu.VMEM((NR // _NW,), jnp.int32),
            pltpu.VMEM((_GW, EMBED_DIM), jnp.float32),
            pltpu.VMEM((_GW, EMBED_DIM), jnp.float32),
            pltpu.VMEM((_GW, EMBED_DIM), jnp.float32),
            pltpu.VMEM((_GW, EMBED_DIM), jnp.float32),
            pltpu.SemaphoreType.DMA,
            pltpu.SemaphoreType.DMA,
            pltpu.SemaphoreType.DMA,
            pltpu.SemaphoreType.DMA,
            pltpu.SemaphoreType.DMA,
            pltpu.SemaphoreType.DMA,
            pltpu.SemaphoreType.DMA,
            pltpu.SemaphoreType.DMA,
        ],
    )
    def k(ue_hbm, ie_hbm, uid_hbm, iid_hbm, ibu_hbm, ibi_hbm,
          o_ue, o_ie, o_ur, o_ir,
          idxs, rows0, rows1, rows2, rows3,
          g0, g1, g2, g3, w0, w1, w2, w3):
        wid = lax.axis_index("s") * _NC + lax.axis_index("c")
        rows = (rows0, rows1, rows2, rows3)
        gsems = (g0, g1, g2, g3)
        wsems = (w0, w1, w2, w3)

        def job(idx_hbm, table_hbm, out_hbm, n):
            per_w = n // _NW
            chunks = per_w // _GW
            base0 = wid * per_w
            pltpu.sync_copy(idx_hbm.at[pl.ds(base0, per_w)],
                            idxs.at[pl.ds(0, per_w)])

            @pl.loop(0, chunks // 4)
            def _(t):
                gs = []
                for v in range(4):
                    off = (4 * t + v) * _GW
                    gs.append(pltpu.async_copy(
                        table_hbm.at[idxs.at[pl.ds(off, _GW)]],
                        rows[v], gsems[v]))
                ws = []
                for v in range(4):
                    off = base0 + (4 * t + v) * _GW
                    gs[v].wait()
                    ws.append(pltpu.async_copy(
                        rows[v], out_hbm.at[pl.ds(off, _GW)], wsems[v]))
                for v in range(4):
                    ws[v].wait()

        def id_job(idx_hbm, table_hbm, out_hbm):
            base = wid * (B // _NW)
            pltpu.sync_copy(idx_hbm.at[pl.ds(base, _GW)],
                            idxs.at[pl.ds(0, _GW)])
            pltpu.async_copy(table_hbm.at[idxs.at[pl.ds(0, _GW)]],
                             rows0, g0).wait()
            pltpu.sync_copy(rows0, out_hbm.at[pl.ds(base, _GW)])

        id_job(uid_hbm, ue_hbm, o_ue)
        id_job(iid_hbm, ie_hbm, o_ie)
        job(ibu_hbm, ie_hbm, o_ur, NR)
        job(ibi_hbm, ue_hbm, o_ir, NR)

    return k(user_embed, item_embed, uid, iid, ibu, ibi)


def _softmax_rows(counts):
    e = jnp.exp(jnp.log(counts + 1.0) / TAU)
    return e / jnp.sum(e, axis=1, keepdims=True)


def _stage1_body(cu1, cu2, ci1, ci2, uemb, iemb, gur, gir, ibu, ibi,
                 ie0, ue0, um, im, uV, iV, W1, b1, o_ref):
    mVu = _dot(um[...], uV[...])
    mVi = _dot(im[...], iV[...])

    u1 = _dot(_softmax_rows(cu1[...]), mVi)
    u2 = _dot(_softmax_rows(cu2[...]), mVu)
    i1 = _dot(_softmax_rows(ci1[...]), mVu)
    i2 = _dot(_softmax_rows(ci2[...]), mVi)

    n0u = jnp.sum((ibu[...][:, :RECENT] == 0).astype(jnp.float32),
                  axis=1)[:, None]
    n0i = jnp.sum((ibi[...][:, :RECENT] == 0).astype(jnp.float32),
                  axis=1)[:, None]
    gu = gur[...].reshape(BATCH_BLK, _PAD // 2, 2 * EMBED_DIM)
    gi = gir[...].reshape(BATCH_BLK, _PAD // 2, 2 * EMBED_DIM)
    su2 = jnp.sum(gu[:, :RECENT // 2, :], axis=1)
    si2 = jnp.sum(gi[:, :RECENT // 2, :], axis=1)
    s_ur = su2[:, :EMBED_DIM] + su2[:, EMBED_DIM:] - n0u * ie0[...]
    s_ir = si2[:, :EMBED_DIM] + si2[:, EMBED_DIM:] - n0i * ue0[...]
    urw = _dot(s_ur, iV[...])
    irw = _dot(s_ir, uV[...])

    h1 = (_dot(uemb[...], W1[0:64, :])
          + _dot(u1, W1[64:192, :])
          + _dot(u2, W1[192:320, :])
          + _dot(urw, W1[320:448, :])
          + _dot(iemb[...], W1[448:512, :])
          + _dot(i1, W1[512:640, :])
          + _dot(i2, W1[640:768, :])
          + _dot(irw, W1[768:896, :])
          + b1[...])
    o_ref[...] = h1


def _stage1(cu1, cu2, ci1, ci2, uemb, iemb, gur, gir, ibu, ibi,
            ie0, ue0, um, im, uV, iV, W1, b1):
    B = cu1.shape[0]
    nblk = B // BATCH_BLK
    D1 = W1.shape[1]
    blk = lambda s: pl.BlockSpec(s, lambda i: (i,) + (0,) * (len(s) - 1))
    full2 = lambda a: pl.BlockSpec(a.shape, lambda i: (0,) * a.ndim)
    g_spec = pl.BlockSpec((BATCH_BLK * _PAD // 2, 2 * EMBED_DIM),
                          lambda i: (i, 0))
    return pl.pallas_call(
        _stage1_body,
        grid=(nblk,),
        in_specs=[
            blk((BATCH_BLK, I_GROUP)), blk((BATCH_BLK, U_GROUP)),
            blk((BATCH_BLK, U_GROUP)), blk((BATCH_BLK, I_GROUP)),
            blk((BATCH_BLK, EMBED_DIM)), blk((BATCH_BLK, EMBED_DIM)),
            g_spec, g_spec,
            blk((BATCH_BLK, _PAD)), blk((BATCH_BLK, _PAD)),
            full2(ie0), full2(ue0), full2(um), full2(im),
            full2(uV), full2(iV), full2(W1), full2(b1),
        ],
        out_specs=blk((BATCH_BLK, D1)),
        out_shape=jax.ShapeDtypeStruct((B, D1), jnp.float32),
    )(cu1, cu2, ci1, ci2, uemb, iemb, gur, gir, ibu, ibi,
      ie0, ue0, um, im, uV, iV, W1, b1)


def _dice(h, alpha):
    n = h.shape[0]
    avg = jnp.mean(h, axis=0, keepdims=True)
    d = h - avg
    std = jnp.sqrt(jnp.sum(d * d, axis=0, keepdims=True) / (n - 1))
    p = jax.nn.sigmoid(d / std + 1e-08)
    return h * p + alpha * h * (1 - p)


def _stage2_body(h1, a1, W2, b2, a2, W3, b3, o_ref):
    h = _dice(h1[...], a1[0, 0])
    h = _dice(_dot(h, W2[...]) + b2[...], a2[0, 0])
    o_ref[...] = jax.nn.sigmoid(_dot(h, W3[...]) + b3[...])


def _stage2(h1, a1, W2, b2, a2, W3, b3):
    B = h1.shape[0]
    return pl.pallas_call(
        _stage2_body,
        out_shape=jax.ShapeDtypeStruct((B, 1), jnp.float32),
    )(h1, a1, W2, b2, a2, W3, b3)


def kernel(x, user_embed, item_embed, u_macro_embed, i_macro_embed,
           uQ, uK, uV, iQ, iK, iV, W1, b1, alpha1, W2, b2, alpha2, W3, b3):
    f32 = jnp.float32
    x = x.astype(jnp.int32)
    uid = x[:, 0]
    cu1 = x[:, 1:1 + I_GROUP].astype(f32)
    cu2 = x[:, 1 + I_GROUP:1 + I_GROUP + U_GROUP].astype(f32)
    ur = x[:, 1 + I_GROUP + U_GROUP:1 + I_GROUP + U_GROUP + RECENT]
    ic = 1 + I_GROUP + U_GROUP + RECENT
    iid = x[:, ic]
    ci1 = x[:, ic + 1:ic + 1 + U_GROUP].astype(f32)
    ci2 = x[:, ic + 1 + U_GROUP:ic + 1 + U_GROUP + I_GROUP].astype(f32)
    ir = x[:, ic + 1 + U_GROUP + I_GROUP:]

    B = x.shape[0]
    npad = _PAD - RECENT
    ibu = jnp.concatenate([ur, ur[:, :npad]], axis=1)
    ibi = jnp.concatenate([ir, ir[:, :npad]], axis=1)

    uemb, iemb, g_ur, g_ir = _sc_gather(user_embed, item_embed, uid, iid,
                                        ibu.reshape(-1), ibi.reshape(-1))
    g_ur = g_ur.reshape(-1, 2 * EMBED_DIM)
    g_ir = g_ir.reshape(-1, 2 * EMBED_DIM)

    h1 = _stage1(cu1, cu2, ci1, ci2, uemb, iemb, g_ur, g_ir, ibu, ibi,
                 item_embed[0:1, :], user_embed[0:1, :],
                 u_macro_embed, i_macro_embed, uV, iV,
                 W1, b1.reshape(1, -1))
    return _stage2(h1, alpha1.reshape(1, 1), W2, b2.reshape(1, -1),
                   alpha2.reshape(1, 1), W3, b3.reshape(1, 1))


# speedup vs baseline: 8.5883x; 1.0291x over previous
import functools
import math

import jax
import jax.numpy as jnp
from jax import lax
from jax.experimental import pallas as pl
from jax.experimental.pallas import tpu as pltpu
from jax.experimental.pallas import tpu_sc as plsc

EMBED_DIM = 64
HIDDEN_DIM = 128
U_GROUP = 101
I_GROUP = 101
RECENT = 50
TAU = 0.8
BATCH_BLK = 256

_NC = 2
_NS = 16
_NW = _NC * _NS
_GW = 128
_PAD = 64
_SPS = 8

_HIGH = jax.lax.Precision.HIGHEST


def _dot(a, b):
    return jnp.dot(a, b, preferred_element_type=jnp.float32)


@jax.jit
def _sc_gather(user_embed, item_embed, uid, iid, ibu, ibi):
    B = uid.shape[0]
    NR = ibu.shape[0]
    mesh = plsc.VectorSubcoreMesh(core_axis_name="c", subcore_axis_name="s")
    out_type = (
        jax.ShapeDtypeStruct((B, EMBED_DIM), jnp.float32),
        jax.ShapeDtypeStruct((B, EMBED_DIM), jnp.float32),
        jax.ShapeDtypeStruct((NR, EMBED_DIM), jnp.float32),
        jax.ShapeDtypeStruct((NR, EMBED_DIM), jnp.float32),
    )

    @functools.partial(
        pl.kernel,
        mesh=mesh,
        out_type=out_type,
        compiler_params=pltpu.CompilerParams(use_tc_tiling_on_sc=False),
        scratch_types=[
            pltpu.VMEM((NR // _NW,), jnp.int32),
            pltpu.VMEM((_GW, EMBED_DIM), jnp.float32),
            pltpu.VMEM((_GW, EMBED_DIM), jnp.float32),
            pltpu.VMEM((_GW, EMBED_DIM), jnp.float32),
            pltpu.VMEM((_GW, EMBED_DIM), jnp.float32),
            pltpu.VMEM((_GW, EMBED_DIM), jnp.float32),
            pltpu.VMEM((_GW, EMBED_DIM), jnp.float32),
            pltpu.VMEM((_GW, EMBED_DIM), jnp.float32),
            pltpu.VMEM((_GW, EMBED_DIM), jnp.float32),
            pltpu.SemaphoreType.DMA,
            pltpu.SemaphoreType.DMA,
            pltpu.SemaphoreType.DMA,
            pltpu.SemaphoreType.DMA,
            pltpu.SemaphoreType.DMA,
            pltpu.SemaphoreType.DMA,
            pltpu.SemaphoreType.DMA,
            pltpu.SemaphoreType.DMA,
            pltpu.SemaphoreType.DMA,
            pltpu.SemaphoreType.DMA,
            pltpu.SemaphoreType.DMA,
            pltpu.SemaphoreType.DMA,
            pltpu.SemaphoreType.DMA,
            pltpu.SemaphoreType.DMA,
            pltpu.SemaphoreType.DMA,
            pltpu.SemaphoreType.DMA,
        ],
    )
    def k(ue_hbm, ie_hbm, uid_hbm, iid_hbm, ibu_hbm, ibi_hbm,
          o_ue, o_ie, o_ur, o_ir,
          idxs, rows0, rows1, rows2, rows3, rows4, rows5, rows6, rows7,
          g0, g1, g2, g3, g4, g5, g6, g7,
          w0, w1, w2, w3, w4, w5, w6, w7):
        wid = lax.axis_index("s") * _NC + lax.axis_index("c")
        rows = (rows0, rows1, rows2, rows3, rows4, rows5, rows6, rows7)
        gsems = (g0, g1, g2, g3, g4, g5, g6, g7)
        wsems = (w0, w1, w2, w3, w4, w5, w6, w7)

        def job(idx_hbm, table_hbm, out_hbm, n):
            per_w = n // _NW
            chunks = per_w // _GW
            base0 = wid * per_w
            pltpu.sync_copy(idx_hbm.at[pl.ds(base0, per_w)],
                            idxs.at[pl.ds(0, per_w)])

            @pl.loop(0, chunks // 8)
            def _(t):
                gs = []
                for v in range(8):
                    off = (8 * t + v) * _GW
                    gs.append(pltpu.async_copy(
                        table_hbm.at[idxs.at[pl.ds(off, _GW)]],
                        rows[v], gsems[v]))
                ws = []
                for v in range(8):
                    off = base0 + (8 * t + v) * _GW
                    gs[v].wait()
                    ws.append(pltpu.async_copy(
                        rows[v], out_hbm.at[pl.ds(off, _GW)], wsems[v]))
                for v in range(8):
                    ws[v].wait()

        def id_job(idx_hbm, table_hbm, out_hbm):
            base = wid * (B // _NW)
            pltpu.sync_copy(idx_hbm.at[pl.ds(base, _GW)],
                            idxs.at[pl.ds(0, _GW)])
            pltpu.async_copy(table_hbm.at[idxs.at[pl.ds(0, _GW)]],
                             rows0, g0).wait()
            pltpu.sync_copy(rows0, out_hbm.at[pl.ds(base, _GW)])

        id_job(uid_hbm, ue_hbm, o_ue)
        id_job(iid_hbm, ie_hbm, o_ie)
        job(ibu_hbm, ie_hbm, o_ur, NR)
        job(ibi_hbm, ue_hbm, o_ir, NR)

    return k(user_embed, item_embed, uid, iid, ibu, ibi)


def _softmax_rows(counts):
    e = jnp.exp(jnp.log(counts + 1.0) / TAU)
    return e / jnp.sum(e, axis=1, keepdims=True)


def _stage1_body(cu1, cu2, ci1, ci2, uemb, iemb, gur, gir, ibu, ibi,
                 ie0, ue0, um, im, uV, iV, W1, b1, o_ref):
    mVu = _dot(um[...], uV[...])
    mVi = _dot(im[...], iV[...])

    u1 = _dot(_softmax_rows(cu1[...]), mVi)
    u2 = _dot(_softmax_rows(cu2[...]), mVu)
    i1 = _dot(_softmax_rows(ci1[...]), mVu)
    i2 = _dot(_softmax_rows(ci2[...]), mVi)

    n0u = jnp.sum((ibu[...][:, :RECENT] == 0).astype(jnp.float32),
                  axis=1)[:, None]
    n0i = jnp.sum((ibi[...][:, :RECENT] == 0).astype(jnp.float32),
                  axis=1)[:, None]
    gu = gur[...].reshape(BATCH_BLK, _PAD // 2, 2 * EMBED_DIM)
    gi = gir[...].reshape(BATCH_BLK, _PAD // 2, 2 * EMBED_DIM)
    su2 = jnp.sum(gu[:, :RECENT // 2, :], axis=1)
    si2 = jnp.sum(gi[:, :RECENT // 2, :], axis=1)
    s_ur = su2[:, :EMBED_DIM] + su2[:, EMBED_DIM:] - n0u * ie0[...]
    s_ir = si2[:, :EMBED_DIM] + si2[:, EMBED_DIM:] - n0i * ue0[...]
    urw = _dot(s_ur, iV[...])
    irw = _dot(s_ir, uV[...])

    h1 = (_dot(uemb[...], W1[0:64, :])
          + _dot(u1, W1[64:192, :])
          + _dot(u2, W1[192:320, :])
          + _dot(urw, W1[320:448, :])
          + _dot(iemb[...], W1[448:512, :])
          + _dot(i1, W1[512:640, :])
          + _dot(i2, W1[640:768, :])
          + _dot(irw, W1[768:896, :])
          + b1[...])
    o_ref[...] = h1


def _stage1(cu1, cu2, ci1, ci2, uemb, iemb, gur, gir, ibu, ibi,
            ie0, ue0, um, im, uV, iV, W1, b1):
    B = cu1.shape[0]
    nblk = B // BATCH_BLK
    D1 = W1.shape[1]
    blk = lambda s: pl.BlockSpec(s, lambda i: (i,) + (0,) * (len(s) - 1))
    full2 = lambda a: pl.BlockSpec(a.shape, lambda i: (0,) * a.ndim)
    g_spec = pl.BlockSpec((BATCH_BLK * _PAD // 2, 2 * EMBED_DIM),
                          lambda i: (i, 0))
    return pl.pallas_call(
        _stage1_body,
        grid=(nblk,),
        in_specs=[
            blk((BATCH_BLK, I_GROUP)), blk((BATCH_BLK, U_GROUP)),
            blk((BATCH_BLK, U_GROUP)), blk((BATCH_BLK, I_GROUP)),
            blk((BATCH_BLK, EMBED_DIM)), blk((BATCH_BLK, EMBED_DIM)),
            g_spec, g_spec,
            blk((BATCH_BLK, _PAD)), blk((BATCH_BLK, _PAD)),
            full2(ie0), full2(ue0), full2(um), full2(im),
            full2(uV), full2(iV), full2(W1), full2(b1),
        ],
        out_specs=blk((BATCH_BLK, D1)),
        out_shape=jax.ShapeDtypeStruct((B, D1), jnp.float32),
    )(cu1, cu2, ci1, ci2, uemb, iemb, gur, gir, ibu, ibi,
      ie0, ue0, um, im, uV, iV, W1, b1)


def _dice(h, alpha):
    n = h.shape[0]
    avg = jnp.mean(h, axis=0, keepdims=True)
    d = h - avg
    std = jnp.sqrt(jnp.sum(d * d, axis=0, keepdims=True) / (n - 1))
    p = jax.nn.sigmoid(d / std + 1e-08)
    return h * p + alpha * h * (1 - p)


def _stage2_body(h1, a1, W2, b2, a2, W3, b3, o_ref):
    h = _dice(h1[...], a1[0, 0])
    h = _dice(_dot(h, W2[...]) + b2[...], a2[0, 0])
    o_ref[...] = jax.nn.sigmoid(_dot(h, W3[...]) + b3[...])


def _stage2(h1, a1, W2, b2, a2, W3, b3):
    B = h1.shape[0]
    return pl.pallas_call(
        _stage2_body,
        out_shape=jax.ShapeDtypeStruct((B, 1), jnp.float32),
    )(h1, a1, W2, b2, a2, W3, b3)


def kernel(x, user_embed, item_embed, u_macro_embed, i_macro_embed,
           uQ, uK, uV, iQ, iK, iV, W1, b1, alpha1, W2, b2, alpha2, W3, b3):
    f32 = jnp.float32
    x = x.astype(jnp.int32)
    uid = x[:, 0]
    cu1 = x[:, 1:1 + I_GROUP].astype(f32)
    cu2 = x[:, 1 + I_GROUP:1 + I_GROUP + U_GROUP].astype(f32)
    ur = x[:, 1 + I_GROUP + U_GROUP:1 + I_GROUP + U_GROUP + RECENT]
    ic = 1 + I_GROUP + U_GROUP + RECENT
    iid = x[:, ic]
    ci1 = x[:, ic + 1:ic + 1 + U_GROUP].astype(f32)
    ci2 = x[:, ic + 1 + U_GROUP:ic + 1 + U_GROUP + I_GROUP].astype(f32)
    ir = x[:, ic + 1 + U_GROUP + I_GROUP:]

    B = x.shape[0]
    npad = _PAD - RECENT
    ibu = jnp.concatenate([ur, ur[:, :npad]], axis=1)
    ibi = jnp.concatenate([ir, ir[:, :npad]], axis=1)

    uemb, iemb, g_ur, g_ir = _sc_gather(user_embed, item_embed, uid, iid,
                                        ibu.reshape(-1), ibi.reshape(-1))
    g_ur = g_ur.reshape(-1, 2 * EMBED_DIM)
    g_ir = g_ir.reshape(-1, 2 * EMBED_DIM)

    h1 = _stage1(cu1, cu2, ci1, ci2, uemb, iemb, g_ur, g_ir, ibu, ibi,
                 item_embed[0:1, :], user_embed[0:1, :],
                 u_macro_embed, i_macro_embed, uV, iV,
                 W1, b1.reshape(1, -1))
    return _stage2(h1, alpha1.reshape(1, 1), W2, b2.reshape(1, -1),
                   alpha2.reshape(1, 1), W3, b3.reshape(1, 1))
